# baseline (device time: 692813 ns/iter reference)
import jax
import jax.numpy as jnp
from jax import lax
from jax.experimental import pallas as pl
from jax.experimental.pallas import tpu as pltpu

jax.config.update("jax_compilation_cache_dir", "/tmp/jaxcache_scband")
jax.config.update("jax_persistent_cache_min_compile_time_secs", 1.0)

N_DEV = 4
M = 8192
K = 2048
N = 4096
HALF = N // 2
CHUNK = M // N_DEV
RCH = CHUNK // 2
STRIP = 256
N_ROUND = 2


def _body(x_ref, w_ref, out_ref,
          buf_a0, buf_b0, buf_a1, buf_b1, work0, work1,
          xbuf0, xbuf1, wh0, wh1, f32st,
          send_sems, recv_sems, load_sems, store_sems, credit0, credit1):
    my = lax.axis_index("i")
    left = (my - 1) % N_DEV
    right = (my + 1) % N_DEV

    work = (work0, work1)
    xbuf = (xbuf0, xbuf1)
    wh = (wh0, wh1)
    credit = (credit0, credit1)
    bufs = [[buf_a0, buf_b0], [buf_a1, buf_b1]]
    send_to = (right, left)
    recv_frm = (left, right)

    def start_send(d):
        r = pltpu.make_async_remote_copy(
            src_ref=bufs[d][0], dst_ref=work[d],
            send_sem=send_sems.at[d], recv_sem=recv_sems.at[d],
            device_id=(send_to[d],), device_id_type=pl.DeviceIdType.MESH)
        r.start()
        return r

    def grant(d):
        pl.semaphore_signal(credit[d], inc=1, device_id=(recv_frm[d],),
                            device_id_type=pl.DeviceIdType.MESH)

    def load_x(rows_base, d):
        for strip in range(RCH // STRIP):
            ld = pltpu.make_async_copy(
                x_ref.at[pl.ds(rows_base + strip * STRIP, STRIP), :],
                f32st, load_sems.at[0])
            ld.start()
            ld.wait()
            xbuf[d][pl.ds(strip * STRIP, STRIP), :] = (
                f32st[...].astype(jnp.bfloat16))

    def dots_into(tgt_ref, d, src_x):
        tgt_ref[...] = jnp.dot(
            src_x[...], wh[d][...], preferred_element_type=jnp.float32,
        ).astype(jnp.bfloat16)

    pending = {0: [], 1: []}

    def store_from(src_ref, c, r, d):
        if pending[d]:
            pending[d].pop().wait()
        st = pltpu.make_async_copy(
            src_ref,
            out_ref.at[pl.ds(c * CHUNK + r * RCH, RCH),
                       pl.ds(d * HALF, HALF)],
            store_sems.at[d])
        st.start()
        pending[d].append(st)

    barrier = pltpu.get_barrier_semaphore()
    for nbr in (left, right):
        pl.semaphore_signal(barrier, inc=1, device_id=(nbr,),
                            device_id_type=pl.DeviceIdType.MESH)

    for d in (0, 1):
        for strip in range(K // STRIP):
            ld = pltpu.make_async_copy(
                w_ref.at[pl.ds(strip * STRIP, STRIP), pl.ds(d * HALF, HALF)],
                f32st, load_sems.at[0])
            ld.start()
            ld.wait()
            wh[d][pl.ds(strip * STRIP, STRIP), :] = (
                f32st[...].astype(jnp.bfloat16))
    load_x(my * CHUNK, 0)
    dots_into(bufs[0][0], 0, xbuf[0])
    dots_into(bufs[1][0], 1, xbuf[0])

    pl.semaphore_wait(barrier, 2)

    first_send = [True]
    for rnd in range(N_ROUND):
        for s in range(N_DEV - 1):
            if not first_send[0]:
                pl.semaphore_wait(credit0, 1)
                pl.semaphore_wait(credit1, 1)
            first_send[0] = False
            rd = [start_send(0), start_send(1)]
            c_recv = ((my - s - 1) % N_DEV, (my + s + 1) % N_DEV)
            for d in (0, 1):
                load_x(c_recv[d] * CHUNK + rnd * RCH, d)
                dots_into(bufs[d][1], d, xbuf[d])
            for d in (0, 1):
                rd[d].wait()
                bufs[d][1][...] = (
                    bufs[d][1][...].astype(jnp.float32)
                    + work[d][...].astype(jnp.float32)).astype(jnp.bfloat16)
                grant(d)
                bufs[d].reverse()

        own = ((my + 1) % N_DEV, (my - 1) % N_DEV)
        for d in (0, 1):
            y = bufs[d][0][...].astype(jnp.float32)
            bufs[d][0][...] = (y * jax.nn.sigmoid(y)).astype(jnp.bfloat16)

        for s in range(N_DEV - 1):
            pl.semaphore_wait(credit0, 1)
            pl.semaphore_wait(credit1, 1)
            rd = [start_send(0), start_send(1)]
            for d in (0, 1):
                sgn = 1 if d == 0 else -1
                c_out = own[d] if s == 0 else (my - sgn * (s - 1)) % N_DEV
                store_from(bufs[d][0], c_out, rnd, d)
            if rnd == 0 and s == N_DEV - 2:
                load_x(my * CHUNK + RCH, 0)
                for d in (0, 1):
                    if pending[d]:
                        pending[d].pop().wait()
                    dots_into(bufs[d][1], d, xbuf[0])
            for d in (0, 1):
                rd[d].wait()
                sgn = 1 if d == 0 else -1
                if s < N_DEV - 2:
                    bufs[d][1][...] = work[d][...]
                    grant(d)
                    bufs[d].reverse()
                else:
                    c_last = (my - sgn * s) % N_DEV
                    store_from(work[d], c_last, rnd, d)
                    if rnd == 0:
                        pending[d].pop().wait()
                        grant(d)
                        bufs[d].reverse()
    for d in (0, 1):
        while pending[d]:
            pending[d].pop().wait()


def kernel(x, w_mat):
    return pl.pallas_call(
        _body,
        out_shape=jax.ShapeDtypeStruct((M, N), jnp.bfloat16),
        in_specs=[pl.BlockSpec(memory_space=pl.ANY),
                  pl.BlockSpec(memory_space=pl.ANY)],
        out_specs=pl.BlockSpec(memory_space=pl.ANY),
        scratch_shapes=[
            pltpu.VMEM((RCH, HALF), jnp.bfloat16),
            pltpu.VMEM((RCH, HALF), jnp.bfloat16),
            pltpu.VMEM((RCH, HALF), jnp.bfloat16),
            pltpu.VMEM((RCH, HALF), jnp.bfloat16),
            pltpu.VMEM((RCH, HALF), jnp.bfloat16),
            pltpu.VMEM((RCH, HALF), jnp.bfloat16),
            pltpu.VMEM((RCH, K), jnp.bfloat16),
            pltpu.VMEM((RCH, K), jnp.bfloat16),
            pltpu.VMEM((K, HALF), jnp.bfloat16),
            pltpu.VMEM((K, HALF), jnp.bfloat16),
            pltpu.VMEM((STRIP, K), jnp.float32),
            pltpu.SemaphoreType.DMA((2,)),
            pltpu.SemaphoreType.DMA((2,)),
            pltpu.SemaphoreType.DMA((2,)),
            pltpu.SemaphoreType.DMA((2,)),
            pltpu.SemaphoreType.REGULAR,
            pltpu.SemaphoreType.REGULAR,
        ],
        compiler_params=pltpu.CompilerParams(
            collective_id=0,
            vmem_limit_bytes=64 * 1024 * 1024,
        ),
    )(x, w_mat)
